# baseline (speedup 1.0000x reference)
import functools

import jax
import jax.numpy as jnp
from jax import lax
from jax.experimental import pallas as pl
from jax.experimental.pallas import tpu as pltpu
from jax.experimental.pallas import tpu_sc as plsc

_F32 = jnp.float32


def _matmul_bias_relu_kernel(w_ref, p_ref, b_ref, o_ref):
    y = lax.dot_general(w_ref[...], p_ref[...], (((1,), (0,)), ((), ())),
                        preferred_element_type=_F32)
    o_ref[...] = jnp.maximum(y + b_ref[...], 0.0)


def _conv_matmul(w2d, p2d, bias, n_block):
    M, K = w2d.shape
    N = p2d.shape[1]
    w2d = w2d.astype(jnp.bfloat16)
    p2d = p2d.astype(jnp.bfloat16)
    grid = (N // n_block,)
    return pl.pallas_call(
        _matmul_bias_relu_kernel,
        grid=grid,
        in_specs=[
            pl.BlockSpec((M, K), lambda i: (0, 0)),
            pl.BlockSpec((K, n_block), lambda i: (0, i)),
            pl.BlockSpec((M, 1), lambda i: (0, 0)),
        ],
        out_specs=pl.BlockSpec((M, n_block), lambda i: (0, i)),
        out_shape=jax.ShapeDtypeStruct((M, N), _F32),
    )(w2d, p2d, bias)


def _pool_norm_kernel(a_ref, o_ref):
    a = a_ref[...]
    m = jnp.maximum(jnp.maximum(a[0], a[1]), jnp.maximum(a[2], a[3]))
    norm = jnp.sqrt(jnp.sum(m * m, axis=-1, keepdims=True))
    o_ref[...] = m / jnp.maximum(norm, 1e-12)


def _pool_norm(a4):
    _, B, C, F = a4.shape
    return pl.pallas_call(
        _pool_norm_kernel,
        out_shape=jax.ShapeDtypeStruct((B, C, F), _F32),
    )(a4)


def _score_argmin_kernel(cb, xn_ref, c_ref, o_ref):
    i = pl.program_id(0)
    xb = xn_ref[...]
    cblk = c_ref[...]
    B = xb.shape[0]
    K = cblk.shape[1]
    cols = []
    for j in range(cb):
        cj = cblk[j]
        cn = jnp.sum(cj * cj, axis=1).reshape(1, K)
        d = lax.dot_general(xb[:, j, :], cj, (((1,), (1,)), ((), ())),
                            preferred_element_type=_F32,
                        precision=lax.Precision.HIGHEST)
        s = cn - 2.0 * d
        m = jnp.min(s, axis=1, keepdims=True)
        iota = lax.broadcasted_iota(jnp.int32, (B, K), 1)
        idx = jnp.min(jnp.where(s == m, iota, K), axis=1, keepdims=True)
        cols.append(idx + (i * cb + j) * K)
    o_ref[...] = jnp.concatenate(cols, axis=1)[None]


def _score_argmin(xn, centers, cb):
    B, C, F = xn.shape
    K = centers.shape[1]
    grid = (C // cb,)
    out3 = pl.pallas_call(
        functools.partial(_score_argmin_kernel, cb),
        grid=grid,
        in_specs=[
            pl.BlockSpec((B, cb, F), lambda i: (0, i, 0)),
            pl.BlockSpec((cb, K, F), lambda i: (i, 0, 0)),
        ],
        out_specs=pl.BlockSpec((1, B, cb), lambda i: (i, 0, 0)),
        out_shape=jax.ShapeDtypeStruct((C // cb, B, cb), jnp.int32),
    )(xn, centers)
    return out3.transpose(1, 0, 2).reshape(B, C)


def _half_select_kernel(w_ref, s_ref, o_ref):
    w = w_ref[...]
    s = s_ref[...][:, :, None]
    o_ref[...] = jnp.where(s == 0, w[:, :, :64], w[:, :, 64:])


def _half_select(wide, parity):
    B, C, _ = wide.shape
    return pl.pallas_call(
        _half_select_kernel,
        out_shape=jax.ShapeDtypeStruct((B, C, 64), _F32),
    )(wide, parity)


def _fc_kernel(x_ref, w1_ref, b1_ref, w2_ref, b2_ref, o_ref):
    i = pl.program_id(0)
    a = lax.dot_general(x_ref[...], w1_ref[...], (((1,), (1,)), ((), ())),
                        preferred_element_type=_F32)
    a = jnp.maximum(a + b1_ref[...], 0.0)
    p = lax.dot_general(a.astype(jnp.bfloat16), w2_ref[...],
                        (((1,), (1,)), ((), ())),
                        preferred_element_type=_F32)

    @pl.when(i == 0)
    def _():
        o_ref[...] = p + b2_ref[...]

    @pl.when(i != 0)
    def _():
        o_ref[...] = o_ref[...] + p


def _fc_head(x, w1, b1, w2, b2, nb):
    B, K = x.shape
    N = w1.shape[0]
    O = w2.shape[0]
    x = x.astype(jnp.bfloat16)
    w1 = w1.astype(jnp.bfloat16)
    w2 = w2.astype(jnp.bfloat16)
    grid = (N // nb,)
    return pl.pallas_call(
        _fc_kernel,
        grid=grid,
        in_specs=[
            pl.BlockSpec((B, K), lambda i: (0, 0)),
            pl.BlockSpec((nb, K), lambda i: (i, 0)),
            pl.BlockSpec((1, nb), lambda i: (0, i)),
            pl.BlockSpec((O, nb), lambda i: (0, i)),
            pl.BlockSpec((1, O), lambda i: (0, 0)),
        ],
        out_specs=pl.BlockSpec((B, O), lambda i: (0, 0)),
        out_shape=jax.ShapeDtypeStruct((B, O), _F32),
    )(x, w1, b1.reshape(1, N), w2, b2.reshape(1, O))


_SC_NC = 2
_SC_NS = 16


def _sc_gather(table, idx):
    V, D = table.shape
    B = idx.shape[0]
    nw = _SC_NC * _SC_NS
    b_per_w = B // nw
    mesh = plsc.VectorSubcoreMesh(core_axis_name="c", subcore_axis_name="s")

    @functools.partial(
        pl.kernel,
        out_type=jax.ShapeDtypeStruct((B, D), _F32),
        mesh=mesh,
        scratch_types=[
            pltpu.VMEM((b_per_w,), jnp.int32),
            pltpu.VMEM((b_per_w, D), _F32),
            pltpu.SemaphoreType.DMA,
        ],
    )
    def k(table_hbm, idx_hbm, out_hbm, idx_v, rows_v, sem):
        wid = lax.axis_index("s") * _SC_NC + lax.axis_index("c")
        base = wid * b_per_w
        pltpu.sync_copy(idx_hbm.at[pl.ds(base, b_per_w)], idx_v)
        pltpu.async_copy(table_hbm.at[idx_v], rows_v, sem).wait()
        pltpu.sync_copy(rows_v, out_hbm.at[pl.ds(base, b_per_w)])

    return k(table, idx)


def _im2col(x, ksize, pad):
    B, C, H, W = x.shape
    xp = jnp.pad(x, ((0, 0), (0, 0), (pad, pad), (pad, pad)))
    taps = [xp[:, :, dy:dy + H, dx:dx + W]
            for dy in range(ksize) for dx in range(ksize)]
    s = jnp.stack(taps)
    s = s.transpose(2, 0, 1, 3, 4)
    return s.reshape(C * ksize * ksize, B * H * W)


def _pool_taps(y, C, B, H):
    Hp = H // 2
    a = y.reshape(C, B, Hp, 2, Hp, 2).transpose(3, 5, 1, 0, 2, 4)
    return a.reshape(4, B, C, Hp * Hp)


def kernel(x, conv1_w, conv1_b, conv2_w, conv2_b, fc1_w, fc1_b, fc2_w,
           fc2_b, center0, center1):
    B = x.shape[0]

    p0 = _im2col(x, 5, 2)
    p0 = jnp.pad(p0, ((0, 5), (0, 0)))
    w1 = jnp.pad(conv1_w.reshape(96, 75), ((0, 0), (0, 5)))
    y1 = _conv_matmul(w1, p0, conv1_b.reshape(96, 1), 2048)

    xn0 = _pool_norm(_pool_taps(y1, 96, B, 32))

    idx0 = _score_argmin(xn0, center0, 8)
    q0 = _sc_gather(center0.reshape(96 * 512, 256), idx0.reshape(-1))
    h0 = q0.reshape(B, 96, 16, 16)

    p1 = _im2col(h0, 5, 2)
    w2 = conv2_w.reshape(192, 2400)
    y2 = _conv_matmul(w2, p1, conv2_b.reshape(192, 1), 512)

    xn1 = _pool_norm(_pool_taps(y2, 192, B, 16))

    idx1 = _score_argmin(xn1, center1, 16)
    wide = _sc_gather(center1.reshape(192 * 512 // 2, 128),
                      (idx1 >> 1).reshape(-1))
    q1 = _half_select(wide.reshape(B, 192, 128), idx1 & 1)

    h1 = q1.reshape(B, 192 * 64)
    return _fc_head(h1, fc1_w, fc1_b, fc2_w, fc2_b, 256)


# speedup vs baseline: 1.1557x; 1.1557x over previous
import functools

import jax
import jax.numpy as jnp
from jax import lax
from jax.experimental import pallas as pl
from jax.experimental.pallas import tpu as pltpu
from jax.experimental.pallas import tpu_sc as plsc

_F32 = jnp.float32


def _matmul_bias_relu_kernel(w_ref, p_ref, b_ref, o_ref):
    y = lax.dot_general(w_ref[...].astype(jnp.bfloat16),
                        p_ref[...].astype(jnp.bfloat16),
                        (((1,), (0,)), ((), ())),
                        preferred_element_type=_F32)
    o_ref[...] = jnp.maximum(y + b_ref[...], 0.0)


def _conv_matmul(w2d, p2d, bias, n_block):
    M, K = w2d.shape
    N = p2d.shape[1]
    grid = (N // n_block,)
    return pl.pallas_call(
        _matmul_bias_relu_kernel,
        grid=grid,
        in_specs=[
            pl.BlockSpec((M, K), lambda i: (0, 0)),
            pl.BlockSpec((K, n_block), lambda i: (0, i)),
            pl.BlockSpec((M, 1), lambda i: (0, 0)),
        ],
        out_specs=pl.BlockSpec((M, n_block), lambda i: (0, i)),
        out_shape=jax.ShapeDtypeStruct((M, N), _F32),
    )(w2d, p2d, bias)


def _pool_norm_kernel(a_ref, o_ref):
    a = a_ref[...]
    m = jnp.maximum(jnp.maximum(a[0], a[1]), jnp.maximum(a[2], a[3]))
    norm = jnp.sqrt(jnp.sum(m * m, axis=-1, keepdims=True))
    o_ref[...] = m / jnp.maximum(norm, 1e-12)


def _pool_norm(a4):
    _, B, C, F = a4.shape
    return pl.pallas_call(
        _pool_norm_kernel,
        out_shape=jax.ShapeDtypeStruct((B, C, F), _F32),
    )(a4)


def _score_argmin_kernel(cb, xn_ref, c_ref, o_ref):
    i = pl.program_id(0)
    xb = xn_ref[...]
    cblk = c_ref[...]
    B = xb.shape[0]
    K = cblk.shape[1]
    cols = []
    for j in range(cb):
        cj = cblk[j]
        cn = jnp.sum(cj * cj, axis=1).reshape(1, K)
        d = lax.dot_general(xb[:, j, :], cj, (((1,), (1,)), ((), ())),
                            preferred_element_type=_F32,
                        precision=lax.Precision.HIGHEST)
        s = cn - 2.0 * d
        m = jnp.min(s, axis=1, keepdims=True)
        iota = lax.broadcasted_iota(jnp.int32, (B, K), 1)
        idx = jnp.min(jnp.where(s == m, iota, K), axis=1, keepdims=True)
        cols.append(idx + (i * cb + j) * K)
    o_ref[...] = jnp.concatenate(cols, axis=1)[None]


def _score_argmin(xn, centers, cb):
    B, C, F = xn.shape
    K = centers.shape[1]
    grid = (C // cb,)
    out3 = pl.pallas_call(
        functools.partial(_score_argmin_kernel, cb),
        grid=grid,
        in_specs=[
            pl.BlockSpec((B, cb, F), lambda i: (0, i, 0)),
            pl.BlockSpec((cb, K, F), lambda i: (i, 0, 0)),
        ],
        out_specs=pl.BlockSpec((1, B, cb), lambda i: (i, 0, 0)),
        out_shape=jax.ShapeDtypeStruct((C // cb, B, cb), jnp.int32),
    )(xn, centers)
    return out3.transpose(1, 0, 2).reshape(B, C)


def _half_select_kernel(w_ref, s_ref, o_ref):
    w = w_ref[...]
    s = s_ref[...][:, :, None]
    o_ref[...] = jnp.where(s == 0, w[:, :, :64], w[:, :, 64:])


def _half_select(wide, parity):
    B, C, _ = wide.shape
    return pl.pallas_call(
        _half_select_kernel,
        out_shape=jax.ShapeDtypeStruct((B, C, 64), _F32),
    )(wide, parity)


def _fc_kernel(x_ref, w1_ref, b1_ref, w2_ref, b2_ref, o_ref):
    i = pl.program_id(0)
    a = lax.dot_general(x_ref[...].astype(jnp.bfloat16),
                        w1_ref[...].astype(jnp.bfloat16),
                        (((1,), (1,)), ((), ())),
                        preferred_element_type=_F32)
    a = jnp.maximum(a + b1_ref[...], 0.0)
    p = lax.dot_general(a.astype(jnp.bfloat16),
                        w2_ref[...].astype(jnp.bfloat16),
                        (((1,), (1,)), ((), ())),
                        preferred_element_type=_F32)

    @pl.when(i == 0)
    def _():
        o_ref[...] = p + b2_ref[...]

    @pl.when(i != 0)
    def _():
        o_ref[...] = o_ref[...] + p


def _fc_head(x, w1, b1, w2, b2, nb):
    B, K = x.shape
    N = w1.shape[0]
    O = w2.shape[0]
    grid = (N // nb,)
    return pl.pallas_call(
        _fc_kernel,
        grid=grid,
        in_specs=[
            pl.BlockSpec((B, K), lambda i: (0, 0)),
            pl.BlockSpec((nb, K), lambda i: (i, 0)),
            pl.BlockSpec((1, nb), lambda i: (0, i)),
            pl.BlockSpec((O, nb), lambda i: (0, i)),
            pl.BlockSpec((1, O), lambda i: (0, 0)),
        ],
        out_specs=pl.BlockSpec((B, O), lambda i: (0, 0)),
        out_shape=jax.ShapeDtypeStruct((B, O), _F32),
    )(x, w1, b1.reshape(1, N), w2, b2.reshape(1, O))


_SC_NC = 2
_SC_NS = 16


def _sc_gather(table, idx):
    V, D = table.shape
    B = idx.shape[0]
    nw = _SC_NC * _SC_NS
    b_per_w = B // nw
    mesh = plsc.VectorSubcoreMesh(core_axis_name="c", subcore_axis_name="s")

    @functools.partial(
        pl.kernel,
        out_type=jax.ShapeDtypeStruct((B, D), _F32),
        mesh=mesh,
        scratch_types=[
            pltpu.VMEM((b_per_w,), jnp.int32),
            pltpu.VMEM((b_per_w, D), _F32),
            pltpu.SemaphoreType.DMA,
        ],
    )
    def k(table_hbm, idx_hbm, out_hbm, idx_v, rows_v, sem):
        wid = lax.axis_index("s") * _SC_NC + lax.axis_index("c")
        base = wid * b_per_w
        pltpu.sync_copy(idx_hbm.at[pl.ds(base, b_per_w)], idx_v)
        pltpu.async_copy(table_hbm.at[idx_v], rows_v, sem).wait()
        pltpu.sync_copy(rows_v, out_hbm.at[pl.ds(base, b_per_w)])

    return k(table, idx)


def _im2col(x, ksize, pad):
    B, C, H, W = x.shape
    xp = jnp.pad(x, ((0, 0), (0, 0), (pad, pad), (pad, pad)))
    taps = [xp[:, :, dy:dy + H, dx:dx + W]
            for dy in range(ksize) for dx in range(ksize)]
    s = jnp.stack(taps)
    s = s.transpose(2, 0, 1, 3, 4)
    return s.reshape(C * ksize * ksize, B * H * W)


def _pool_taps(y, C, B, H):
    Hp = H // 2
    a = y.reshape(C, B, Hp, 2, Hp, 2).transpose(3, 5, 1, 0, 2, 4)
    return a.reshape(4, B, C, Hp * Hp)


def kernel(x, conv1_w, conv1_b, conv2_w, conv2_b, fc1_w, fc1_b, fc2_w,
           fc2_b, center0, center1):
    B = x.shape[0]

    p0 = _im2col(x, 5, 2)
    p0 = jnp.pad(p0, ((0, 5), (0, 0)))
    w1 = jnp.pad(conv1_w.reshape(96, 75), ((0, 0), (0, 5)))
    y1 = _conv_matmul(w1, p0, conv1_b.reshape(96, 1), 2048)

    xn0 = _pool_norm(_pool_taps(y1, 96, B, 32))

    idx0 = _score_argmin(xn0, center0, 8)
    q0 = _sc_gather(center0.reshape(96 * 512, 256), idx0.reshape(-1))
    h0 = q0.reshape(B, 96, 16, 16)

    p1 = _im2col(h0, 5, 2)
    w2 = conv2_w.reshape(192, 2400)
    y2 = _conv_matmul(w2, p1, conv2_b.reshape(192, 1), 512)

    xn1 = _pool_norm(_pool_taps(y2, 192, B, 16))

    idx1 = _score_argmin(xn1, center1, 16)
    wide = _sc_gather(center1.reshape(192 * 512 // 2, 128),
                      (idx1 >> 1).reshape(-1))
    q1 = _half_select(wide.reshape(B, 192, 128), idx1 & 1)

    h1 = q1.reshape(B, 192 * 64)
    return _fc_head(h1, fc1_w, fc1_b, fc2_w, fc2_b, 256)
